# baseline (speedup 1.0000x reference)
import functools

import jax
import jax.numpy as jnp
from jax import lax
from jax.experimental import pallas as pl
from jax.experimental.pallas import tpu as pltpu
from jax.experimental.pallas import tpu_sc as plsc

N = 10000
D = 768
E = 100000

LANES = 128
NCH = D // LANES
NSUB = 16
NCORES = 2
NREP = NCH // NCORES
N_PAD = ((N + 255) // 256) * 256
E_PAD = ((E + 2047) // 2048) * 2048
BW = 64
E_PAD = NSUB * 99 * BW
BT = E_PAD // (NSUB * BW)
N_ACC = 10112
RPT = N_ACC // NSUB
CW = N_PAD // NSUB
NBUF = 3

_mesh = plsc.VectorSubcoreMesh(core_axis_name="c", subcore_axis_name="s")
_sc_params = pltpu.CompilerParams(needs_layout_passes=False)


def _deg_body(edges_ref, out_ref, idx_v, acc_v, res_v, tmp_v, part_s):
    c = lax.axis_index("c")
    s = lax.axis_index("s")
    pltpu.sync_copy(edges_ref.at[c, s], idx_v)

    zero16 = jnp.zeros((16,), jnp.float32)
    ones16 = jnp.ones((16,), jnp.float32)

    @pl.loop(0, N_PAD // 16)
    def _(i):
        acc_v[pl.ds(pl.multiple_of(i * 16, 16), 16)] = zero16

    @pl.loop(0, BT)
    def _(b):
        for k in range(BW // 16):
            idx = idx_v[b, pl.ds(16 * k, 16)]
            plsc.addupdate_scatter(acc_v, [idx], ones16)

    pltpu.sync_copy(acc_v, part_s.at[s])
    plsc.subcore_barrier()
    for r in range(NSUB):
        pltpu.sync_copy(part_s.at[r, pl.ds(s * CW, CW)], tmp_v.at[r])

    @pl.loop(0, CW // 16)
    def _(i):
        off = pl.ds(pl.multiple_of(i * 16, 16), 16)
        tot = tmp_v[0, off]
        for r in range(1, NSUB):
            tot = tot + tmp_v[r, off]
        res_v[off] = tot

    pltpu.sync_copy(res_v, out_ref.at[c, pl.ds(s * CW, CW)])


@functools.partial(
    pl.kernel,
    out_type=jax.ShapeDtypeStruct((2, N_PAD), jnp.float32),
    mesh=_mesh,
    compiler_params=_sc_params,
    scratch_types=[
        pltpu.VMEM((BT, BW), jnp.int32),
        pltpu.VMEM((N_PAD,), jnp.float32),
        pltpu.VMEM((CW,), jnp.float32),
        pltpu.VMEM((NSUB, CW), jnp.float32),
        pltpu.VMEM_SHARED((NSUB, N_PAD), jnp.float32),
    ],
)
def _degrees(edges_ref, out_ref, idx_v, acc_v, res_v, tmp_v, part_s):
    _deg_body(edges_ref, out_ref, idx_v, acc_v, res_v, tmp_v, part_s)


def _make_edge_scatter():
    @functools.partial(
        pl.kernel,
        out_type=jax.ShapeDtypeStruct((2, N_ACC, LANES), jnp.float32),
        mesh=_mesh,
        compiler_params=_sc_params,
        scratch_types=[
            pltpu.VMEM((BT * BW,), jnp.int32),
            pltpu.VMEM((BT, BW), jnp.int32),
            [pltpu.VMEM((BW, LANES), jnp.float32) for _ in range(NBUF)],
            pltpu.VMEM_SHARED((N_ACC, LANES), jnp.float32),
            [pltpu.SemaphoreType.DMA for _ in range(NBUF)],
            [pltpu.SemaphoreType.DMA for _ in range(NBUF)],
        ],
    )
    def _k(y_ref, src_ref, dst_ref, z_ref, agg_ref,
           sidx_v, didx_v, bufs, acc_s, gsems, ssems):
        c = lax.axis_index("c")
        s = lax.axis_index("s")
        pltpu.sync_copy(src_ref.at[s], sidx_v)
        pltpu.sync_copy(dst_ref.at[s], didx_v)

        pltpu.sync_copy(z_ref, bufs[0])
        for p in range(RPT // BW):
            pltpu.sync_copy(bufs[0], acc_s.at[pl.ds(s * RPT + p * BW, BW)])
        rem = RPT - BW * (RPT // BW)
        if rem:
            pltpu.sync_copy(bufs[0].at[pl.ds(0, rem)],
                            acc_s.at[pl.ds(s * RPT + BW * (RPT // BW), rem)])
        plsc.subcore_barrier()

        for half in range(NCORES):
            @pl.when(c == half)
            def _(half=half):
                tab = y_ref.at[half]

                def sidx(b):
                    return sidx_v.at[pl.ds(pl.multiple_of(b * BW, BW), BW)]

                def fire_g(b, k):
                    pltpu.async_copy(tab.at[sidx(b)], bufs[k], gsems[k])

                def fire_s(b, k):
                    pltpu.async_copy(bufs[k], acc_s.at[didx_v.at[b]],
                                     ssems[k], add=True)

                def drain_g(k):
                    pltpu.make_async_copy(
                        tab.at[sidx(0)], bufs[k], gsems[k]).wait()

                def drain_s(k):
                    pltpu.make_async_copy(
                        bufs[k], acc_s.at[didx_v.at[0]], ssems[k]).wait()

                for k in range(NBUF):
                    fire_g(k, k)

                @pl.loop(0, BT // NBUF - 1)
                def _(j):
                    b0 = j * NBUF
                    for k in range(NBUF):
                        drain_g(k)
                        fire_s(b0 + k, k)
                    for k in range(NBUF):
                        drain_s(k)
                        fire_g(b0 + NBUF + k, k)

                for k in range(NBUF):
                    drain_g(k)
                    fire_s(BT - NBUF + k, k)
                for k in range(NBUF):
                    drain_s(k)

        plsc.subcore_barrier()

        for half in range(NCORES):
            @pl.when(c == half)
            def _(half=half):
                pltpu.sync_copy(
                    acc_s.at[pl.ds(s * RPT, RPT)],
                    agg_ref.at[half, pl.ds(s * RPT, RPT)],
                )

    return _k


_edge_scatter = _make_edge_scatter()


R = 512


def _row_norm(deg_ref):
    i = pl.program_id(0)
    return lax.rsqrt(deg_ref[pl.ds(i * R, R)] + 1.0)


def _mm1_body(x_ref, deg_ref, w_ref, out_ref):
    norm = _row_norm(deg_ref)
    xn = (x_ref[...] * norm[:, None]).astype(jnp.bfloat16)
    out_ref[0] = jnp.dot(xn, w_ref[...].astype(jnp.bfloat16),
                         preferred_element_type=jnp.float32)


def _mm2_body(h0_ref, h1_ref, h2_ref, deg_ref, w_ref, out_ref):
    norm = _row_norm(deg_ref)
    acc = jnp.zeros((R, LANES), jnp.float32)
    for rp, href in enumerate((h0_ref, h1_ref, h2_ref)):
        for kp in range(2):
            hk = (href[kp] * norm[:, None]).astype(jnp.bfloat16)
            acc = acc + jnp.dot(
                hk, w_ref[pl.ds((rp + 3 * kp) * LANES, LANES), :].astype(
                    jnp.bfloat16),
                preferred_element_type=jnp.float32)
    out_ref[0] = acc


def _h1_body(agg_ref, y_ref, deg_ref, b_ref, out_ref):
    norm = _row_norm(deg_ref)
    out_ref[0] = (agg_ref[0] + y_ref[0]) * norm[:, None] + b_ref[0]


def _fin_body(x_ref, h1_ref, agg_ref, y_ref, deg_ref, b_ref, out_ref):
    norm = _row_norm(deg_ref)
    out_ref[...] = (x_ref[...] + h1_ref[0]
                    + (agg_ref[0] + y_ref[0]) * norm[:, None]
                    + b_ref[0])


def _fin_body_prev(x_ref, h1_ref, agg_ref, y_ref, deg_ref, b_ref, prev_ref,
                   out_ref):
    del prev_ref
    _fin_body(x_ref, h1_ref, agg_ref, y_ref, deg_ref, b_ref, out_ref)


_pair = jax.ShapeDtypeStruct((2, N_PAD, LANES), jnp.float32)
_gridp = (N_PAD // R, 2)
_spec_pair = pl.BlockSpec((1, R, LANES), lambda i, k: (k, i, 0))
_spec_aggp = pl.BlockSpec((1, R, LANES), lambda i, k: (k, i, 0))
_spec_deg = pl.BlockSpec((N_PAD,), lambda i, k: (0,))


def _bias_spec(rep):
    return pl.BlockSpec((1, 1, LANES), lambda i, k, rep=rep: (rep + 3 * k, 0, 0))


def _mm1(x, deg, w, rep):
    return pl.pallas_call(
        _mm1_body,
        grid=_gridp,
        in_specs=[
            pl.BlockSpec((R, D), lambda i, k: (i, 0)),
            _spec_deg,
            pl.BlockSpec((D, LANES), lambda i, k: (0, k)),
        ],
        out_specs=_spec_pair,
        out_shape=_pair,
    )(x, deg, w)


def _mm2(h_parts, deg, w, rep):
    return pl.pallas_call(
        _mm2_body,
        grid=_gridp,
        in_specs=[
            pl.BlockSpec((2, R, LANES), lambda i, k: (0, i, 0)),
            pl.BlockSpec((2, R, LANES), lambda i, k: (0, i, 0)),
            pl.BlockSpec((2, R, LANES), lambda i, k: (0, i, 0)),
            _spec_deg,
            pl.BlockSpec((D, LANES), lambda i, k: (0, k)),
        ],
        out_specs=_spec_pair,
        out_shape=_pair,
    )(*h_parts, deg, w)


def _h1(agg, y, deg, b, rep):
    return pl.pallas_call(
        _h1_body,
        grid=_gridp,
        in_specs=[_spec_pair, _spec_pair, _spec_deg, _bias_spec(rep)],
        out_specs=_spec_pair,
        out_shape=_pair,
    )(agg, y, deg, b)


def _fin(x, h1p, agg, y, deg, b, rep, prev):
    def xspec(i, k, rep=rep):
        return (i, rep + 3 * k)

    in_specs = [
        pl.BlockSpec((R, LANES), xspec),
        _spec_pair, _spec_pair, _spec_pair,
        _spec_deg, _bias_spec(rep),
    ]
    args = [x, h1p, agg, y, deg, b]
    aliases = {}
    body = _fin_body
    if prev is not None:
        in_specs.append(pl.BlockSpec(memory_space=pl.ANY))
        args.append(prev)
        aliases = {6: 0}
        body = _fin_body_prev
    return pl.pallas_call(
        body,
        grid=_gridp,
        in_specs=in_specs,
        out_specs=pl.BlockSpec((R, LANES), xspec),
        out_shape=jax.ShapeDtypeStruct((N, D), jnp.float32),
        input_output_aliases=aliases,
    )(*args)


def kernel(x, edge_index, W1, b1, W2, b2):
    pad = E_PAD - E
    fill = jnp.full((pad,), N, jnp.int32)
    srcp = jnp.concatenate([edge_index[0], fill]).reshape(NSUB, BT * BW)
    dstp = jnp.concatenate([edge_index[1], fill]).reshape(NSUB, BT, BW)
    zrows = jnp.zeros((BW, LANES), jnp.float32)
    b1c = b1.reshape(NCH, 1, LANES)
    b2c = b2.reshape(NCH, 1, LANES)
    w1p = [jnp.concatenate([W1[:, r * LANES:(r + 1) * LANES],
                            W1[:, (r + 3) * LANES:(r + 4) * LANES]], axis=1)
           for r in range(NREP)]
    w2p = [jnp.concatenate([W2[:, r * LANES:(r + 1) * LANES],
                            W2[:, (r + 3) * LANES:(r + 4) * LANES]], axis=1)
           for r in range(NREP)]

    deg2 = _degrees(jnp.stack([srcp.reshape(NSUB, BT, BW), dstp]))
    deg_src = deg2[0]
    deg_dst = deg2[1]

    y1 = [_mm1(x, deg_src, w1p[r], r) for r in range(NREP)]
    agg1 = [_edge_scatter(y1[r], srcp, dstp, zrows) for r in range(NREP)]
    h1 = [_h1(agg1[r], y1[r], deg_dst, b1c, r) for r in range(NREP)]
    y2 = [_mm2(h1, deg_src, w2p[r], r) for r in range(NREP)]
    agg2 = [_edge_scatter(y2[r], srcp, dstp, zrows) for r in range(NREP)]

    out = None
    for r in range(NREP):
        out = _fin(x, h1[r], agg2[r], y2[r], deg_dst, b2c, r, out)
    return out


# speedup vs baseline: 1.3016x; 1.3016x over previous
import functools

import jax
import jax.numpy as jnp
from jax import lax
from jax.experimental import pallas as pl
from jax.experimental.pallas import tpu as pltpu
from jax.experimental.pallas import tpu_sc as plsc

N = 10000
D = 768
E = 100000

LANES = 128
NCH = D // LANES
NSUB = 16
NCORES = 2
NREP = NCH // NCORES
N_PAD = ((N + 255) // 256) * 256
E_PAD = ((E + 2047) // 2048) * 2048
EB = E_PAD // LANES
BT = EB // NSUB
RPT = N_PAD // NSUB
CW = N_PAD // NSUB
NBUF = 2

_mesh = plsc.VectorSubcoreMesh(core_axis_name="c", subcore_axis_name="s")
_sc_params = pltpu.CompilerParams(needs_layout_passes=False)


def _deg_body(edges_ref, out_ref, idx_v, acc_v, res_v, tmp_v, part_s):
    c = lax.axis_index("c")
    s = lax.axis_index("s")
    pltpu.sync_copy(edges_ref.at[c, s], idx_v)

    zero16 = jnp.zeros((16,), jnp.float32)
    ones16 = jnp.ones((16,), jnp.float32)

    @pl.loop(0, N_PAD // 16)
    def _(i):
        acc_v[pl.ds(pl.multiple_of(i * 16, 16), 16)] = zero16

    @pl.loop(0, BT)
    def _(b):
        for k in range(LANES // 16):
            idx = idx_v[b, pl.ds(16 * k, 16)]
            plsc.addupdate_scatter(acc_v, [idx], ones16)

    pltpu.sync_copy(acc_v, part_s.at[s])
    plsc.subcore_barrier()
    for r in range(NSUB):
        pltpu.sync_copy(part_s.at[r, pl.ds(s * CW, CW)], tmp_v.at[r])

    @pl.loop(0, CW // 16)
    def _(i):
        off = pl.ds(pl.multiple_of(i * 16, 16), 16)
        tot = tmp_v[0, off]
        for r in range(1, NSUB):
            tot = tot + tmp_v[r, off]
        res_v[off] = tot

    pltpu.sync_copy(res_v, out_ref.at[c, pl.ds(s * CW, CW)])


@functools.partial(
    pl.kernel,
    out_type=jax.ShapeDtypeStruct((2, N_PAD), jnp.float32),
    mesh=_mesh,
    compiler_params=_sc_params,
    scratch_types=[
        pltpu.VMEM((BT, LANES), jnp.int32),
        pltpu.VMEM((N_PAD,), jnp.float32),
        pltpu.VMEM((CW,), jnp.float32),
        pltpu.VMEM((NSUB, CW), jnp.float32),
        pltpu.VMEM_SHARED((NSUB, N_PAD), jnp.float32),
    ],
)
def _degrees(edges_ref, out_ref, idx_v, acc_v, res_v, tmp_v, part_s):
    _deg_body(edges_ref, out_ref, idx_v, acc_v, res_v, tmp_v, part_s)


def _make_edge_scatter():
    @functools.partial(
        pl.kernel,
        out_type=jax.ShapeDtypeStruct((2, N_PAD, LANES), jnp.float32),
        mesh=_mesh,
        compiler_params=_sc_params,
        scratch_types=[
            pltpu.VMEM((BT, LANES), jnp.int32),
            pltpu.VMEM((BT, LANES), jnp.int32),
            [pltpu.VMEM((LANES, LANES), jnp.float32) for _ in range(NBUF)],
            pltpu.VMEM_SHARED((N_PAD, LANES), jnp.float32),
            [pltpu.SemaphoreType.DMA for _ in range(NBUF)],
            [pltpu.SemaphoreType.DMA for _ in range(NBUF)],
        ],
    )
    def _k(y_ref, src_ref, dst_ref, z_ref, agg_ref,
           sidx_v, didx_v, bufs, acc_s, gsems, ssems):
        c = lax.axis_index("c")
        s = lax.axis_index("s")
        pltpu.sync_copy(src_ref.at[s], sidx_v)
        pltpu.sync_copy(dst_ref.at[s], didx_v)

        pltpu.sync_copy(z_ref, bufs[0])
        for p in range(RPT // LANES):
            pltpu.sync_copy(bufs[0], acc_s.at[pl.ds(s * RPT + p * LANES, LANES)])
        plsc.subcore_barrier()

        for half in range(NCORES):
            @pl.when(c == half)
            def _(half=half):
                tab = y_ref.at[half]

                def fire_g(b, k):
                    pltpu.async_copy(tab.at[sidx_v.at[b]], bufs[k], gsems[k])

                def fire_s(b, k):
                    pltpu.async_copy(bufs[k], acc_s.at[didx_v.at[b]],
                                     ssems[k], add=True)

                def drain_g(k):
                    pltpu.make_async_copy(
                        tab.at[sidx_v.at[0]], bufs[k], gsems[k]).wait()

                def drain_s(k):
                    pltpu.make_async_copy(
                        bufs[k], acc_s.at[didx_v.at[0]], ssems[k]).wait()

                fire_g(0, 0)
                fire_g(1, 1)

                @pl.loop(0, BT // 2 - 1)
                def _(j):
                    b = j * 2
                    drain_g(0)
                    fire_s(b, 0)
                    drain_g(1)
                    fire_s(b + 1, 1)
                    drain_s(0)
                    fire_g(b + 2, 0)
                    drain_s(1)
                    fire_g(b + 3, 1)

                b = 2 * (BT // 2 - 1)
                drain_g(0)
                fire_s(b, 0)
                drain_g(1)
                fire_s(b + 1, 1)
                drain_s(0)
                for bb in range(b + 2, BT):
                    fire_g(bb, 0)
                    drain_g(0)
                    fire_s(bb, 0)
                    drain_s(0)
                drain_s(1)

        plsc.subcore_barrier()

        for half in range(NCORES):
            @pl.when(c == half)
            def _(half=half):
                pltpu.sync_copy(
                    acc_s.at[pl.ds(s * RPT, RPT)],
                    agg_ref.at[half, pl.ds(s * RPT, RPT)],
                )

    return _k


_edge_scatter = _make_edge_scatter()


R = 512


def _row_norm(deg_ref):
    i = pl.program_id(0)
    return lax.rsqrt(deg_ref[pl.ds(i * R, R)] + 1.0)


def _mm1_body(x_ref, deg_ref, w_ref, out_ref):
    norm = _row_norm(deg_ref)
    xn = (x_ref[...] * norm[:, None]).astype(jnp.bfloat16)
    out_ref[0] = jnp.dot(xn, w_ref[...].astype(jnp.bfloat16),
                         preferred_element_type=jnp.float32)


def _mm2_body(h0_ref, h1_ref, h2_ref, deg_ref, w_ref, out_ref):
    norm = _row_norm(deg_ref)
    acc = jnp.zeros((R, LANES), jnp.float32)
    for rp, href in enumerate((h0_ref, h1_ref, h2_ref)):
        for kp in range(2):
            hk = (href[kp] * norm[:, None]).astype(jnp.bfloat16)
            acc = acc + jnp.dot(
                hk, w_ref[pl.ds((rp + 3 * kp) * LANES, LANES), :].astype(
                    jnp.bfloat16),
                preferred_element_type=jnp.float32)
    out_ref[0] = acc


def _h1_body(agg_ref, y_ref, deg_ref, b_ref, out_ref):
    norm = _row_norm(deg_ref)
    out_ref[0] = (agg_ref[0] + y_ref[0]) * norm[:, None] + b_ref[0]


def _fin_body(x_ref, h1_ref, agg_ref, y_ref, deg_ref, b_ref, out_ref):
    norm = _row_norm(deg_ref)
    out_ref[...] = (x_ref[...] + h1_ref[0]
                    + (agg_ref[0] + y_ref[0]) * norm[:, None]
                    + b_ref[0])


def _fin_body_prev(x_ref, h1_ref, agg_ref, y_ref, deg_ref, b_ref, prev_ref,
                   out_ref):
    del prev_ref
    _fin_body(x_ref, h1_ref, agg_ref, y_ref, deg_ref, b_ref, out_ref)


_pair = jax.ShapeDtypeStruct((2, N_PAD, LANES), jnp.float32)
_gridp = (N_PAD // R, 2)
_spec_pair = pl.BlockSpec((1, R, LANES), lambda i, k: (k, i, 0))
_spec_deg = pl.BlockSpec((N_PAD,), lambda i, k: (0,))


def _bias_spec(rep):
    return pl.BlockSpec((1, 1, LANES), lambda i, k, rep=rep: (rep + 3 * k, 0, 0))


def _mm1(x, deg, w, rep):
    return pl.pallas_call(
        _mm1_body,
        grid=_gridp,
        in_specs=[
            pl.BlockSpec((R, D), lambda i, k: (i, 0)),
            _spec_deg,
            pl.BlockSpec((D, LANES), lambda i, k: (0, k)),
        ],
        out_specs=_spec_pair,
        out_shape=_pair,
    )(x, deg, w)


def _mm2(h_parts, deg, w, rep):
    return pl.pallas_call(
        _mm2_body,
        grid=_gridp,
        in_specs=[
            pl.BlockSpec((2, R, LANES), lambda i, k: (0, i, 0)),
            pl.BlockSpec((2, R, LANES), lambda i, k: (0, i, 0)),
            pl.BlockSpec((2, R, LANES), lambda i, k: (0, i, 0)),
            _spec_deg,
            pl.BlockSpec((D, LANES), lambda i, k: (0, k)),
        ],
        out_specs=_spec_pair,
        out_shape=_pair,
    )(*h_parts, deg, w)


def _h1(agg, y, deg, b, rep):
    return pl.pallas_call(
        _h1_body,
        grid=_gridp,
        in_specs=[_spec_pair, _spec_pair, _spec_deg, _bias_spec(rep)],
        out_specs=_spec_pair,
        out_shape=_pair,
    )(agg, y, deg, b)


def _fin(x, h1p, agg, y, deg, b, rep, prev):
    def xspec(i, k, rep=rep):
        return (i, rep + 3 * k)

    in_specs = [
        pl.BlockSpec((R, LANES), xspec),
        _spec_pair, _spec_pair, _spec_pair,
        _spec_deg, _bias_spec(rep),
    ]
    args = [x, h1p, agg, y, deg, b]
    aliases = {}
    body = _fin_body
    if prev is not None:
        in_specs.append(pl.BlockSpec(memory_space=pl.ANY))
        args.append(prev)
        aliases = {6: 0}
        body = _fin_body_prev
    return pl.pallas_call(
        body,
        grid=_gridp,
        in_specs=in_specs,
        out_specs=pl.BlockSpec((R, LANES), xspec),
        out_shape=jax.ShapeDtypeStruct((N, D), jnp.float32),
        input_output_aliases=aliases,
    )(*args)


def kernel(x, edge_index, W1, b1, W2, b2):
    pad = E_PAD - E
    fill = jnp.full((pad,), N, jnp.int32)
    srcp = jnp.concatenate([edge_index[0], fill]).reshape(NSUB, BT, LANES)
    dstp = jnp.concatenate([edge_index[1], fill]).reshape(NSUB, BT, LANES)
    zrows = jnp.zeros((LANES, LANES), jnp.float32)
    b1c = b1.reshape(NCH, 1, LANES)
    b2c = b2.reshape(NCH, 1, LANES)
    w1p = [jnp.concatenate([W1[:, r * LANES:(r + 1) * LANES],
                            W1[:, (r + 3) * LANES:(r + 4) * LANES]], axis=1)
           for r in range(NREP)]
    w2p = [jnp.concatenate([W2[:, r * LANES:(r + 1) * LANES],
                            W2[:, (r + 3) * LANES:(r + 4) * LANES]], axis=1)
           for r in range(NREP)]

    deg2 = _degrees(jnp.stack([srcp, dstp]))
    deg_src = deg2[0]
    deg_dst = deg2[1]

    y1 = [_mm1(x, deg_src, w1p[r], r) for r in range(NREP)]
    agg1 = [_edge_scatter(y1[r], srcp, dstp, zrows) for r in range(NREP)]
    h1 = [_h1(agg1[r], y1[r], deg_dst, b1c, r) for r in range(NREP)]
    y2 = [_mm2(h1, deg_src, w2p[r], r) for r in range(NREP)]
    agg2 = [_edge_scatter(y2[r], srcp, dstp, zrows) for r in range(NREP)]

    out = None
    for r in range(NREP):
        out = _fin(x, h1[r], agg2[r], y2[r], deg_dst, b2c, r, out)
    return out
